# baseline (speedup 1.0000x reference)
import functools

import jax
import jax.numpy as jnp
from jax import lax
from jax.experimental import pallas as pl
from jax.experimental.pallas import tpu as pltpu
from jax.experimental.pallas import tpu_sc as plsc

N_NODES = 10000
N_PAD = 10112
D = 128
N_EDGES = 320000
NC, NS = 2, 16
NW = NC * NS
CHUNK = 128
EPT = 10240
E_PAD = NW * EPT
NCHUNK = EPT // CHUNK
ROWS_PT = N_PAD // NS
NBUF = 2
GROUP = 8
NG0 = 18
NG1 = 2
_WB_CHUNKS = -(-ROWS_PT // CHUNK)
E0 = NS * NG0 * GROUP * CHUNK

_MESH = plsc.VectorSubcoreMesh(
    core_axis_name="c", subcore_axis_name="s", num_cores=NC, num_subcores=NS
)

_ZERO16 = functools.partial(jnp.zeros, (16,), jnp.float32)


def _deg_body(dst_hbm, deg_hbm, dst_v, deg_v):
    c = lax.axis_index("c")
    s = lax.axis_index("s")
    wid = c * NS + s
    pltpu.sync_copy(dst_hbm.at[pl.ds(wid * EPT, EPT)], dst_v)

    def zero(i, _):
        deg_v[pl.ds(i * 16, 16)] = _ZERO16()
        return 0

    lax.fori_loop(0, N_PAD // 16, zero, 0)

    ones = jnp.ones((16,), jnp.float32)

    def count(i, _):
        idx = dst_v[pl.ds(i * 16, 16)]
        plsc.addupdate_scatter(deg_v, [idx], ones)
        return 0

    lax.fori_loop(0, EPT // 16, count, 0)
    pltpu.sync_copy(deg_v, deg_hbm.at[pl.ds(wid * N_PAD, N_PAD)])


_SC_PARAMS = pltpu.CompilerParams(needs_layout_passes=False)

_deg_call = pl.kernel(
    _deg_body,
    out_type=jax.ShapeDtypeStruct((NW * N_PAD,), jnp.float32),
    mesh=_MESH,
    compiler_params=_SC_PARAMS,
    scratch_types=[
        pltpu.VMEM((EPT,), jnp.int32),
        pltpu.VMEM((N_PAD,), jnp.float32),
    ],
)


def _run_groups(g_hbm, src_hbm, dst_hbm, acc_sh, rows, sems, src_v, dst_v,
                s, ngroups):
    def group(j, _):
        pltpu.sync_copy(src_hbm.at[s, j], src_v)
        pltpu.sync_copy(dst_hbm.at[s, j], dst_v)
        for b in range(NBUF):
            pltpu.async_copy(g_hbm.at[src_v.at[b]], rows[b], sems[b])
        for k in range(GROUP):
            b = k % NBUF
            pltpu.make_async_copy(
                g_hbm.at[src_v.at[k]], rows[b], sems[b]
            ).wait()
            pltpu.sync_copy(rows[b], acc_sh.at[dst_v.at[k]], add=True)
            if k + NBUF < GROUP:
                pltpu.async_copy(
                    g_hbm.at[src_v.at[k + NBUF]], rows[b], sems[b]
                )
        return 0

    lax.fori_loop(0, ngroups, group, 0)


def _fill_row_ids(dst_v, base):
    lanes = lax.iota(jnp.int32, 16)

    def fill(i, _):
        e = i * 16 + lanes
        dst_v[i // 8, pl.ds((i % 8) * 16, 16)] = base + jnp.minimum(
            e, ROWS_PT - 1
        )
        return 0

    lax.fori_loop(0, GROUP * CHUNK // 16, fill, 0)


def _agg_body(g_hbm, srcA_hbm, dstA_hbm, srcB_hbm, dstB_hbm, out_hbm,
              src_v, dst_v, r0, r1, acc_sh, s0, s1):
    rows = (r0, r1)
    sems = (s0, s1)
    c = lax.axis_index("c")
    s = lax.axis_index("s")
    base = s * ROWS_PT

    def zero(i, _):
        r0[i // 8, pl.ds((i % 8) * 16, 16)] = _ZERO16()
        return 0

    lax.fori_loop(0, CHUNK * (D // 16), zero, 0)

    @pl.when((c == 0) | (NG1 > 0))
    def _():
        _fill_row_ids(dst_v, base)
        for k in range(_WB_CHUNKS):
            pltpu.sync_copy(r0, acc_sh.at[dst_v.at[k]])

    plsc.subcore_barrier()

    @pl.when(c == 0)
    def _():
        _run_groups(g_hbm, srcA_hbm, dstA_hbm, acc_sh, rows, sems,
                    src_v, dst_v, s, NG0)

    @pl.when(c == 1)
    def _():
        _run_groups(g_hbm, srcB_hbm, dstB_hbm, acc_sh, rows, sems,
                    src_v, dst_v, s, NG1)

    plsc.subcore_barrier()

    @pl.when(c == 0)
    def _():
        _fill_row_ids(dst_v, base)
        for b in range(NBUF):
            pltpu.async_copy(acc_sh.at[dst_v.at[b]], rows[b], sems[b])
        for k in range(_WB_CHUNKS):
            b = k % NBUF
            n = CHUNK if (k + 1) * CHUNK <= ROWS_PT else ROWS_PT - k * CHUNK
            pltpu.make_async_copy(
                acc_sh.at[dst_v.at[k]], rows[b], sems[b]
            ).wait()
            pltpu.sync_copy(
                rows[b].at[pl.ds(0, n)],
                out_hbm.at[0, pl.ds(base + k * CHUNK, n)],
            )
            if k + NBUF < _WB_CHUNKS:
                pltpu.async_copy(
                    acc_sh.at[dst_v.at[k + NBUF]], rows[b], sems[b]
                )

    @pl.when((c == 1) & (NG1 > 0))
    def _():
        _fill_row_ids(dst_v, base)
        for b in range(NBUF):
            pltpu.async_copy(acc_sh.at[dst_v.at[b]], rows[b], sems[b])
        for k in range(_WB_CHUNKS):
            b = k % NBUF
            n = CHUNK if (k + 1) * CHUNK <= ROWS_PT else ROWS_PT - k * CHUNK
            pltpu.make_async_copy(
                acc_sh.at[dst_v.at[k]], rows[b], sems[b]
            ).wait()
            pltpu.sync_copy(
                rows[b].at[pl.ds(0, n)],
                out_hbm.at[1, pl.ds(base + k * CHUNK, n)],
            )
            if k + NBUF < _WB_CHUNKS:
                pltpu.async_copy(
                    acc_sh.at[dst_v.at[k + NBUF]], rows[b], sems[b]
                )

    @pl.when((c == 1) & (NG1 == 0))
    def _():
        for k in range(_WB_CHUNKS):
            n = CHUNK if (k + 1) * CHUNK <= ROWS_PT else ROWS_PT - k * CHUNK
            pltpu.sync_copy(
                r0.at[pl.ds(0, n)],
                out_hbm.at[1, pl.ds(base + k * CHUNK, n)],
            )


_agg_call = pl.kernel(
    _agg_body,
    out_type=jax.ShapeDtypeStruct((NC, N_PAD, D), jnp.float32),
    mesh=_MESH,
    compiler_params=_SC_PARAMS,
    scratch_types=[
        pltpu.VMEM((GROUP, CHUNK), jnp.int32),
        pltpu.VMEM((GROUP, CHUNK), jnp.int32),
        pltpu.VMEM((CHUNK, D), jnp.float32),
        pltpu.VMEM((CHUNK, D), jnp.float32),
        pltpu.VMEM_SHARED((N_PAD, D), jnp.float32),
        pltpu.SemaphoreType.DMA,
        pltpu.SemaphoreType.DMA,
    ],
)


_R = 1000


def _dinv(deg_ref):
    return lax.rsqrt(jnp.sum(deg_ref[...], axis=1, keepdims=True) + 1.0)


def _tc1_body(deg_ref, x_ref, w_ref, g_ref):
    h = jnp.dot(x_ref[...], w_ref[...], preferred_element_type=jnp.float32)
    g_ref[...] = _dinv(deg_ref) * h


def _tc2_body(deg_ref, agg_ref, g1_ref, w_ref, b_ref, g2_ref):
    dinv = _dinv(deg_ref)
    out1 = dinv * (agg_ref[0] + agg_ref[1] + g1_ref[...]) + b_ref[...]
    g2_ref[...] = dinv * jnp.dot(
        out1, w_ref[...], preferred_element_type=jnp.float32
    )


def _tc3_body(deg_ref, agg_ref, g2_ref, b_ref, out_ref):
    dinv = _dinv(deg_ref)
    pre = dinv * (agg_ref[0] + agg_ref[1] + g2_ref[...]) + b_ref[...]
    out_ref[...] = jnp.maximum(pre, 0.0)


_deg_spec = pl.BlockSpec((_R, NW), lambda i: (i, 0))
_row_spec = pl.BlockSpec((_R, D), lambda i: (i, 0))
_agg_spec = pl.BlockSpec((NC, _R, D), lambda i: (0, i, 0))
_w_spec = pl.BlockSpec((D, D), lambda i: (0, 0))
_b_spec = pl.BlockSpec((1, D), lambda i: (0, 0))
_out_struct = jax.ShapeDtypeStruct((N_NODES, D), jnp.float32)

_tc1 = pl.pallas_call(
    _tc1_body,
    grid=(N_NODES // _R,),
    in_specs=[_deg_spec, _row_spec, _w_spec],
    out_specs=_row_spec,
    out_shape=_out_struct,
)
_tc2 = pl.pallas_call(
    _tc2_body,
    grid=(N_NODES // _R,),
    in_specs=[_deg_spec, _agg_spec, _row_spec, _w_spec, _b_spec],
    out_specs=_row_spec,
    out_shape=_out_struct,
)
_tc3 = pl.pallas_call(
    _tc3_body,
    grid=(N_NODES // _R,),
    in_specs=[_deg_spec, _agg_spec, _row_spec, _b_spec],
    out_specs=_row_spec,
    out_shape=_out_struct,
)


def kernel(x, edge_index, W1, b1, W2, b2):
    src = edge_index[0].astype(jnp.int32)
    dst = edge_index[1].astype(jnp.int32)
    pad = E_PAD - N_EDGES
    src_p = jnp.concatenate([src, jnp.zeros((pad,), jnp.int32)])
    dst_p = jnp.concatenate([dst, jnp.full((pad,), N_NODES, jnp.int32)])
    srcA = src_p[:E0].reshape(NS, NG0, GROUP, CHUNK)
    dstA = dst_p[:E0].reshape(NS, NG0, GROUP, CHUNK)
    if NG1 > 0:
        srcB = src_p[E0:].reshape(NS, NG1, GROUP, CHUNK)
        dstB = dst_p[E0:].reshape(NS, NG1, GROUP, CHUNK)
    else:
        srcB = srcA
        dstB = dstA

    degT = _deg_call(dst_p).reshape(NW, N_PAD).T
    g1 = _tc1(degT, x, W1)
    agg1 = _agg_call(g1, srcA, dstA, srcB, dstB)
    g2 = _tc2(degT, agg1, g1, W2, b1.reshape(1, D))
    agg2 = _agg_call(g2, srcA, dstA, srcB, dstB)
    return _tc3(degT, agg2, g2, b2.reshape(1, D))


# speedup vs baseline: 1.0241x; 1.0241x over previous
import functools

import jax
import jax.numpy as jnp
from jax import lax
from jax.experimental import pallas as pl
from jax.experimental.pallas import tpu as pltpu
from jax.experimental.pallas import tpu_sc as plsc

N_NODES = 10000
N_PAD = 10112
D = 128
N_EDGES = 320000
NC, NS = 2, 16
NW = NC * NS
CHUNK = 128
EPT = 10240
E_PAD = NW * EPT
NCHUNK = EPT // CHUNK
ROWS_PT = N_PAD // NS
NBUF = 2
GROUP = 8
NG0 = 19
NG1 = 1
_WB_CHUNKS = -(-ROWS_PT // CHUNK)
E0 = NS * NG0 * GROUP * CHUNK

_MESH = plsc.VectorSubcoreMesh(
    core_axis_name="c", subcore_axis_name="s", num_cores=NC, num_subcores=NS
)

_ZERO16 = functools.partial(jnp.zeros, (16,), jnp.float32)


def _deg_body(dst_hbm, deg_hbm, dst_v, deg_v):
    c = lax.axis_index("c")
    s = lax.axis_index("s")
    wid = c * NS + s
    pltpu.sync_copy(dst_hbm.at[pl.ds(wid * EPT, EPT)], dst_v)

    def zero(i, _):
        deg_v[pl.ds(i * 16, 16)] = _ZERO16()
        return 0

    lax.fori_loop(0, N_PAD // 16, zero, 0)

    ones = jnp.ones((16,), jnp.float32)

    def count(i, _):
        idx = dst_v[pl.ds(i * 16, 16)]
        plsc.addupdate_scatter(deg_v, [idx], ones)
        return 0

    lax.fori_loop(0, EPT // 16, count, 0)
    pltpu.sync_copy(deg_v, deg_hbm.at[pl.ds(wid * N_PAD, N_PAD)])


_SC_PARAMS = pltpu.CompilerParams(needs_layout_passes=False)

_deg_call = pl.kernel(
    _deg_body,
    out_type=jax.ShapeDtypeStruct((NW * N_PAD,), jnp.float32),
    mesh=_MESH,
    compiler_params=_SC_PARAMS,
    scratch_types=[
        pltpu.VMEM((EPT,), jnp.int32),
        pltpu.VMEM((N_PAD,), jnp.float32),
    ],
)


def _run_groups(g_hbm, src_hbm, dst_hbm, acc_sh, rows, sems, src_v, dst_v,
                s, ngroups):
    def group(j, _):
        pltpu.sync_copy(src_hbm.at[s, j], src_v)
        pltpu.sync_copy(dst_hbm.at[s, j], dst_v)
        for b in range(NBUF):
            pltpu.async_copy(g_hbm.at[src_v.at[b]], rows[b], sems[b])
        for k in range(GROUP):
            b = k % NBUF
            pltpu.make_async_copy(
                g_hbm.at[src_v.at[k]], rows[b], sems[b]
            ).wait()
            pltpu.sync_copy(rows[b], acc_sh.at[dst_v.at[k]], add=True)
            if k + NBUF < GROUP:
                pltpu.async_copy(
                    g_hbm.at[src_v.at[k + NBUF]], rows[b], sems[b]
                )
        return 0

    lax.fori_loop(0, ngroups, group, 0)


def _fill_row_ids(dst_v, base):
    lanes = lax.iota(jnp.int32, 16)

    def fill(i, _):
        e = i * 16 + lanes
        dst_v[i // 8, pl.ds((i % 8) * 16, 16)] = base + jnp.minimum(
            e, ROWS_PT - 1
        )
        return 0

    lax.fori_loop(0, GROUP * CHUNK // 16, fill, 0)


def _agg_body(g_hbm, srcA_hbm, dstA_hbm, srcB_hbm, dstB_hbm, out_hbm,
              src_v, dst_v, r0, r1, acc_sh, s0, s1):
    rows = (r0, r1)
    sems = (s0, s1)
    c = lax.axis_index("c")
    s = lax.axis_index("s")
    base = s * ROWS_PT

    def zero(i, _):
        r0[i // 8, pl.ds((i % 8) * 16, 16)] = _ZERO16()
        return 0

    lax.fori_loop(0, CHUNK * (D // 16), zero, 0)

    @pl.when((c == 0) | (NG1 > 0))
    def _():
        _fill_row_ids(dst_v, base)
        for k in range(_WB_CHUNKS):
            pltpu.sync_copy(r0, acc_sh.at[dst_v.at[k]])

    plsc.subcore_barrier()

    @pl.when(c == 0)
    def _():
        _run_groups(g_hbm, srcA_hbm, dstA_hbm, acc_sh, rows, sems,
                    src_v, dst_v, s, NG0)

    @pl.when(c == 1)
    def _():
        _run_groups(g_hbm, srcB_hbm, dstB_hbm, acc_sh, rows, sems,
                    src_v, dst_v, s, NG1)

    plsc.subcore_barrier()

    @pl.when(c == 0)
    def _():
        _fill_row_ids(dst_v, base)
        for b in range(NBUF):
            pltpu.async_copy(acc_sh.at[dst_v.at[b]], rows[b], sems[b])
        for k in range(_WB_CHUNKS):
            b = k % NBUF
            n = CHUNK if (k + 1) * CHUNK <= ROWS_PT else ROWS_PT - k * CHUNK
            pltpu.make_async_copy(
                acc_sh.at[dst_v.at[k]], rows[b], sems[b]
            ).wait()
            pltpu.sync_copy(
                rows[b].at[pl.ds(0, n)],
                out_hbm.at[0, pl.ds(base + k * CHUNK, n)],
            )
            if k + NBUF < _WB_CHUNKS:
                pltpu.async_copy(
                    acc_sh.at[dst_v.at[k + NBUF]], rows[b], sems[b]
                )

    @pl.when((c == 1) & (NG1 > 0))
    def _():
        _fill_row_ids(dst_v, base)
        for b in range(NBUF):
            pltpu.async_copy(acc_sh.at[dst_v.at[b]], rows[b], sems[b])
        for k in range(_WB_CHUNKS):
            b = k % NBUF
            n = CHUNK if (k + 1) * CHUNK <= ROWS_PT else ROWS_PT - k * CHUNK
            pltpu.make_async_copy(
                acc_sh.at[dst_v.at[k]], rows[b], sems[b]
            ).wait()
            pltpu.sync_copy(
                rows[b].at[pl.ds(0, n)],
                out_hbm.at[1, pl.ds(base + k * CHUNK, n)],
            )
            if k + NBUF < _WB_CHUNKS:
                pltpu.async_copy(
                    acc_sh.at[dst_v.at[k + NBUF]], rows[b], sems[b]
                )

    @pl.when((c == 1) & (NG1 == 0))
    def _():
        for k in range(_WB_CHUNKS):
            n = CHUNK if (k + 1) * CHUNK <= ROWS_PT else ROWS_PT - k * CHUNK
            pltpu.sync_copy(
                r0.at[pl.ds(0, n)],
                out_hbm.at[1, pl.ds(base + k * CHUNK, n)],
            )


_agg_call = pl.kernel(
    _agg_body,
    out_type=jax.ShapeDtypeStruct((NC, N_PAD, D), jnp.float32),
    mesh=_MESH,
    compiler_params=_SC_PARAMS,
    scratch_types=[
        pltpu.VMEM((GROUP, CHUNK), jnp.int32),
        pltpu.VMEM((GROUP, CHUNK), jnp.int32),
        pltpu.VMEM((CHUNK, D), jnp.float32),
        pltpu.VMEM((CHUNK, D), jnp.float32),
        pltpu.VMEM_SHARED((N_PAD, D), jnp.float32),
        pltpu.SemaphoreType.DMA,
        pltpu.SemaphoreType.DMA,
    ],
)


_R = 1000


def _dinv(deg_ref):
    return lax.rsqrt(jnp.sum(deg_ref[...], axis=1, keepdims=True) + 1.0)


def _tc1_body(deg_ref, x_ref, w_ref, g_ref):
    h = jnp.dot(x_ref[...], w_ref[...], preferred_element_type=jnp.float32)
    g_ref[...] = _dinv(deg_ref) * h


def _tc2_body(deg_ref, agg_ref, g1_ref, w_ref, b_ref, g2_ref):
    dinv = _dinv(deg_ref)
    out1 = dinv * (agg_ref[0] + agg_ref[1] + g1_ref[...]) + b_ref[...]
    g2_ref[...] = dinv * jnp.dot(
        out1, w_ref[...], preferred_element_type=jnp.float32
    )


def _tc3_body(deg_ref, agg_ref, g2_ref, b_ref, out_ref):
    dinv = _dinv(deg_ref)
    pre = dinv * (agg_ref[0] + agg_ref[1] + g2_ref[...]) + b_ref[...]
    out_ref[...] = jnp.maximum(pre, 0.0)


_deg_spec = pl.BlockSpec((_R, NW), lambda i: (i, 0))
_row_spec = pl.BlockSpec((_R, D), lambda i: (i, 0))
_agg_spec = pl.BlockSpec((NC, _R, D), lambda i: (0, i, 0))
_w_spec = pl.BlockSpec((D, D), lambda i: (0, 0))
_b_spec = pl.BlockSpec((1, D), lambda i: (0, 0))
_out_struct = jax.ShapeDtypeStruct((N_NODES, D), jnp.float32)

_tc1 = pl.pallas_call(
    _tc1_body,
    grid=(N_NODES // _R,),
    in_specs=[_deg_spec, _row_spec, _w_spec],
    out_specs=_row_spec,
    out_shape=_out_struct,
)
_tc2 = pl.pallas_call(
    _tc2_body,
    grid=(N_NODES // _R,),
    in_specs=[_deg_spec, _agg_spec, _row_spec, _w_spec, _b_spec],
    out_specs=_row_spec,
    out_shape=_out_struct,
)
_tc3 = pl.pallas_call(
    _tc3_body,
    grid=(N_NODES // _R,),
    in_specs=[_deg_spec, _agg_spec, _row_spec, _b_spec],
    out_specs=_row_spec,
    out_shape=_out_struct,
)


def kernel(x, edge_index, W1, b1, W2, b2):
    src = edge_index[0].astype(jnp.int32)
    dst = edge_index[1].astype(jnp.int32)
    pad = E_PAD - N_EDGES
    src_p = jnp.concatenate([src, jnp.zeros((pad,), jnp.int32)])
    dst_p = jnp.concatenate([dst, jnp.full((pad,), N_NODES, jnp.int32)])
    srcA = src_p[:E0].reshape(NS, NG0, GROUP, CHUNK)
    dstA = dst_p[:E0].reshape(NS, NG0, GROUP, CHUNK)
    if NG1 > 0:
        srcB = src_p[E0:].reshape(NS, NG1, GROUP, CHUNK)
        dstB = dst_p[E0:].reshape(NS, NG1, GROUP, CHUNK)
    else:
        srcB = srcA
        dstB = dstA

    degT = _deg_call(dst_p).reshape(NW, N_PAD).T
    g1 = _tc1(degT, x, W1)
    agg1 = _agg_call(g1, srcA, dstA, srcB, dstB)
    g2 = _tc2(degT, agg1, g1, W2, b1.reshape(1, D))
    agg2 = _agg_call(g2, srcA, dstA, srcB, dstB)
    return _tc3(degT, agg2, g2, b2.reshape(1, D))
